# baseline (speedup 1.0000x reference)
import jax
import jax.numpy as jnp
from jax import lax
from jax.experimental import pallas as pl
from jax.experimental.pallas import tpu as pltpu
from jax.experimental.pallas import tpu_sc as plsc

EMB = 64
LANES = 16


def _sc_lookup(idx_hbm, half_hbm, table_hbm, out_hbm, idx_v, half_v, row_v,
               out_v, sem):
    wid = lax.axis_index("s") * 2 + lax.axis_index("c")

    @pl.when(wid == 0)
    def _():
        pltpu.sync_copy(idx_hbm, idx_v)
        pltpu.sync_copy(half_hbm, half_v)
        cp = pltpu.async_copy(table_hbm.at[idx_v], row_v, sem)
        h = half_v[:][0]
        off = pl.multiple_of(h * EMB, LANES)
        cp.wait()
        for k in range(EMB // LANES):
            chunk = row_v[0, pl.ds(off + k * LANES, LANES)]
            out_v[0, pl.ds(k * LANES, LANES)] = chunk
            out_v[0, pl.ds(EMB + k * LANES, LANES)] = chunk
        pltpu.sync_copy(out_v, out_hbm)


def kernel(table, token):
    rows, emb = table.shape
    pair_view = table.reshape(rows // 2, 2 * emb)
    tok = jnp.asarray(token, jnp.int32)
    idx = (tok // 2).reshape(1)
    half = jnp.broadcast_to((tok % 2).reshape(1), (LANES,))
    out = pl.kernel(
        _sc_lookup,
        out_type=jax.ShapeDtypeStruct((1, 2 * EMB), jnp.float32),
        mesh=plsc.VectorSubcoreMesh(core_axis_name="c", subcore_axis_name="s"),
        scratch_types=[
            pltpu.VMEM((1,), jnp.int32),
            pltpu.VMEM((LANES,), jnp.int32),
            pltpu.VMEM((1, 2 * EMB), jnp.float32),
            pltpu.VMEM((1, 2 * EMB), jnp.float32),
            pltpu.SemaphoreType.DMA,
        ],
    )(idx, half, pair_view)
    return out[0, :EMB]


# speedup vs baseline: 1.7295x; 1.7295x over previous
import jax
import jax.numpy as jnp
from jax import lax
from jax.experimental import pallas as pl
from jax.experimental.pallas import tpu as pltpu
from jax.experimental.pallas import tpu_sc as plsc

EMB = 64
LANES = 16


def _sc_lookup(idx_hbm, table_hbm, out_hbm, idx_v, row_v, out_v, sem):
    wid = lax.axis_index("s") * 2 + lax.axis_index("c")

    @pl.when(wid == 0)
    def _():
        pltpu.sync_copy(idx_hbm, idx_v)
        i = idx_v[:][0]
        pltpu.sync_copy(table_hbm.at[pl.ds(i, 1), :], row_v)
        for k in range(EMB // LANES):
            chunk = row_v[0, pl.ds(k * LANES, LANES)]
            out_v[0, pl.ds(k * LANES, LANES)] = chunk
            out_v[0, pl.ds(EMB + k * LANES, LANES)] = chunk
        pltpu.sync_copy(out_v, out_hbm)


def kernel(table, token):
    idx = jnp.broadcast_to(jnp.asarray(token, jnp.int32).reshape(1), (LANES,))
    out = pl.kernel(
        _sc_lookup,
        out_type=jax.ShapeDtypeStruct((1, 2 * EMB), jnp.float32),
        mesh=plsc.VectorSubcoreMesh(core_axis_name="c", subcore_axis_name="s"),
        scratch_types=[
            pltpu.VMEM((LANES,), jnp.int32),
            pltpu.VMEM((1, EMB), jnp.float32),
            pltpu.VMEM((1, 2 * EMB), jnp.float32),
            pltpu.SemaphoreType.DMA,
        ],
    )(idx, table)
    return out[0, :EMB]
